# baseline (speedup 1.0000x reference)
import jax
import jax.numpy as jnp
from jax.experimental import pallas as pl

N = 1024
D_IN = 512
D_OUT = 64


def _gcn_body(x_ref, a_ref, w_ref, b_ref, o_ref):
    sup = jnp.dot(x_ref[:], w_ref[:].astype(jnp.bfloat16),
                  preferred_element_type=jnp.float32)
    o_ref[:] = jnp.dot(a_ref[:], sup.astype(jnp.bfloat16),
                       preferred_element_type=jnp.float32) + b_ref[:]


def kernel(input, adj, weight, bias):
    return pl.pallas_call(
        _gcn_body,
        out_shape=jax.ShapeDtypeStruct((N, D_OUT), jnp.float32),
    )(input.astype(jnp.bfloat16), adj.astype(jnp.bfloat16),
      weight, bias.reshape(1, D_OUT))


# speedup vs baseline: 1.2565x; 1.2565x over previous
import jax
import jax.numpy as jnp
from jax.experimental import pallas as pl
from jax.experimental.pallas import tpu as pltpu

N = 1024
D_IN = 512
D_OUT = 64
BLK = 128


def _gcn_body(x_ref, a_ref, w_ref, b_ref, o_ref, sup_ref):
    i = pl.program_id(0)

    @pl.when(i == 0)
    def _init():
        xb = x_ref[:].astype(jnp.bfloat16)
        wb = w_ref[:].astype(jnp.bfloat16)
        sup_ref[:] = jnp.dot(xb, wb, preferred_element_type=jnp.float32).astype(
            jnp.bfloat16
        )

    o_ref[:] = jnp.dot(
        a_ref[:].astype(jnp.bfloat16), sup_ref[:], preferred_element_type=jnp.float32
    ) + b_ref[:]


def kernel(input, adj, weight, bias):
    return pl.pallas_call(
        _gcn_body,
        grid=(N // BLK,),
        in_specs=[
            pl.BlockSpec((N, D_IN), lambda i: (0, 0)),
            pl.BlockSpec((BLK, N), lambda i: (i, 0)),
            pl.BlockSpec((D_IN, D_OUT), lambda i: (0, 0)),
            pl.BlockSpec((1, D_OUT), lambda i: (0, 0)),
        ],
        out_specs=pl.BlockSpec((BLK, D_OUT), lambda i: (i, 0)),
        out_shape=jax.ShapeDtypeStruct((N, D_OUT), jnp.float32),
        scratch_shapes=[pltpu.VMEM((N, D_OUT), jnp.bfloat16)],
    )(input, adj, weight, bias.reshape(1, D_OUT))


# speedup vs baseline: 1.6401x; 1.3053x over previous
import jax
import jax.numpy as jnp
from jax.experimental import pallas as pl

N = 1024
D_IN = 512
D_OUT = 64


def _gcn_body(x_ref, a_ref, w_ref, b_ref, o_ref):
    sup = jnp.dot(x_ref[:], w_ref[:], preferred_element_type=jnp.float32)
    o_ref[:] = jnp.dot(a_ref[:], sup, preferred_element_type=jnp.float32) + b_ref[:]


def kernel(input, adj, weight, bias):
    return pl.pallas_call(
        _gcn_body,
        out_shape=jax.ShapeDtypeStruct((N, D_OUT), jnp.float32),
    )(input, adj, weight, bias.reshape(1, D_OUT))
